# baseline (speedup 1.0000x reference)
import dataclasses
import functools

import jax
import jax.numpy as jnp
from jax import lax
from jax.experimental import pallas as pl
from jax.experimental.pallas import tpu as pltpu
from jax.experimental.pallas import tpu_sc as plsc

N_BOND = 50000
N_ETYPES = 36
E_PER_TYPE = 2048
BATCH = 64
FEAT = 768
N_EDGES = N_ETYPES * E_PER_TYPE
N_SLOT = 128
N_ACC = BATCH * N_SLOT

ROWS_BLK = 7168
N_BLKS = 7
N_PAD = ROWS_BLK * N_BLKS


def _matvec_body(h_ref, w_ref, out_ref):
    out_ref[...] = jnp.sum(h_ref[...] * w_ref[...], axis=1)


def _matvec(h, w):
    return pl.pallas_call(
        _matvec_body,
        grid=(N_BLKS,),
        in_specs=[
            pl.BlockSpec((ROWS_BLK, FEAT), lambda i: (i, 0)),
            pl.BlockSpec((1, FEAT), lambda i: (0, 0)),
        ],
        out_specs=pl.BlockSpec((ROWS_BLK,), lambda i: (i,)),
        out_shape=jax.ShapeDtypeStruct((N_PAD,), jnp.float32),
    )(h, w)


NC = 2
NS = 16
NW = NC * NS
CHUNK = N_EDGES // NW
SUB = 256
NSUB = CHUNK // SUB
LANES = 16


@functools.cache
def _get_sc_kernel():
    mesh = plsc.VectorSubcoreMesh(core_axis_name="c", subcore_axis_name="s")
    cp = pltpu.CompilerParams()
    if "needs_layout_passes" in pltpu.CompilerParams.__dataclass_fields__:
        cp = dataclasses.replace(cp, needs_layout_passes=False)
    return functools.partial(
        pl.kernel,
        out_type=jax.ShapeDtypeStruct((NW, N_ACC), jnp.float32),
        mesh=mesh,
        scratch_types=[
            pltpu.VMEM((N_PAD,), jnp.float32),
            pltpu.VMEM((CHUNK,), jnp.int32),
            pltpu.VMEM((CHUNK,), jnp.int32),
            pltpu.VMEM((N_ACC,), jnp.float32),
            pltpu.SemaphoreType.DMA,
            pltpu.SemaphoreType.DMA,
        ],
        compiler_params=cp,
    )(_sc_gather_segsum_body)


def _sc_gather_segsum_body(hw_hbm, src_hbm, seg_hbm, out_hbm,
                           hw_v, src_v, seg_v, acc_v, sem_hw, sem_edge):
    wid = lax.axis_index("s") * NC + lax.axis_index("c")
    base = wid * CHUNK
    cp_hw = pltpu.async_copy(hw_hbm, hw_v, sem_hw)
    edge_cps = []
    for j in range(NSUB):
        g = wid * NSUB + j
        row = g // (E_PER_TYPE // SUB)
        col = (g % (E_PER_TYPE // SUB)) * SUB
        edge_cps.append(pltpu.async_copy(
            src_hbm.at[row, pl.ds(col, SUB)], src_v.at[pl.ds(j * SUB, SUB)],
            sem_edge))
        edge_cps.append(pltpu.async_copy(
            seg_hbm.at[row, pl.ds(col, SUB)], seg_v.at[pl.ds(j * SUB, SUB)],
            sem_edge))

    zeros = jnp.zeros((LANES,), jnp.float32)

    @pl.loop(0, N_ACC, step=LANES)
    def _(i):
        acc_v[pl.ds(i, LANES)] = zeros

    for cp in edge_cps:
        cp.wait()
    cp_hw.wait()

    lane = lax.iota(jnp.int32, LANES)
    shift = jnp.minimum(lane + 1, LANES - 1)
    last = lane == LANES - 1
    notlast = lane < LANES - 1

    @plsc.parallel_loop(0, CHUNK, step=LANES, unroll=8)
    def _(i):
        idx = src_v[pl.ds(i, LANES)]
        segv = seg_v[pl.ds(i, LANES)]
        seg_next = plsc.load_gather(seg_v, [i + shift])
        vals = plsc.load_gather(hw_v, [idx])
        cs = plsc.cumsum(vals)
        t = (base + i) // E_PER_TYPE
        ids = segv * N_SLOT + t
        ids_next = seg_next * N_SLOT + t
        is_end = jnp.logical_or(segv != seg_next, last)
        m2 = jnp.logical_and(is_end, notlast)
        plsc.addupdate_scatter(acc_v, [ids], cs, mask=is_end)
        plsc.addupdate_scatter(acc_v, [ids_next], -cs, mask=m2)

    pltpu.sync_copy(acc_v, out_hbm.at[wid])


def _finalize_body(p_ref, m_ref, o_ref):
    s = p_ref[0]
    for k in range(1, NW):
        s = s + p_ref[k]
    sm = s.reshape(BATCH, N_SLOT)[:, :N_ETYPES]
    masked = jnp.where(m_ref[...] != 0, jnp.float32(-1000000000.0), sm)
    mx = jnp.max(masked, axis=1, keepdims=True)
    e = jnp.exp(masked - mx)
    o_ref[...] = e / jnp.sum(e, axis=1, keepdims=True)


def _finalize(partials, maskf):
    return pl.pallas_call(
        _finalize_body,
        out_shape=jax.ShapeDtypeStruct((BATCH, N_ETYPES), jnp.float32),
    )(partials, maskf)


@jax.jit
def _impl(h, edge_src, seg_ids, maskf, W):
    hw = _matvec(h, W)
    src = edge_src.astype(jnp.int32)
    seg = seg_ids.astype(jnp.int32)
    partials = _get_sc_kernel()(hw, src, seg)
    return _finalize(partials, maskf)


def kernel(h, edge_src, seg_ids, mask_mat, W):
    return _impl(h, edge_src, seg_ids, mask_mat.astype(jnp.float32), W)


# speedup vs baseline: 1.0095x; 1.0095x over previous
import dataclasses
import functools

import jax
import jax.numpy as jnp
from jax import lax
from jax.experimental import pallas as pl
from jax.experimental.pallas import tpu as pltpu
from jax.experimental.pallas import tpu_sc as plsc

N_BOND = 50000
N_ETYPES = 36
E_PER_TYPE = 2048
BATCH = 64
FEAT = 768
N_EDGES = N_ETYPES * E_PER_TYPE
N_SLOT = 128
N_ACC = BATCH * N_SLOT

ROWS_BLK = 5120
N_BLKS = 10
N_PAD = ROWS_BLK * N_BLKS


def _matvec_body(h_ref, w_ref, out_ref):
    out_ref[...] = jnp.sum(h_ref[...] * w_ref[...], axis=1)


def _matvec(h, w):
    return pl.pallas_call(
        _matvec_body,
        grid=(N_BLKS,),
        in_specs=[
            pl.BlockSpec((ROWS_BLK, FEAT), lambda i: (i, 0)),
            pl.BlockSpec((1, FEAT), lambda i: (0, 0)),
        ],
        out_specs=pl.BlockSpec((ROWS_BLK,), lambda i: (i,)),
        out_shape=jax.ShapeDtypeStruct((N_PAD,), jnp.float32),
    )(h, w)


NC = 2
NS = 16
NW = NC * NS
CHUNK = N_EDGES // NW
SUB = 256
NSUB = CHUNK // SUB
LANES = 16


@functools.cache
def _get_sc_kernel():
    mesh = plsc.VectorSubcoreMesh(core_axis_name="c", subcore_axis_name="s")
    cp = pltpu.CompilerParams()
    if "needs_layout_passes" in pltpu.CompilerParams.__dataclass_fields__:
        cp = dataclasses.replace(cp, needs_layout_passes=False)
    return functools.partial(
        pl.kernel,
        out_type=jax.ShapeDtypeStruct((NC, N_ACC), jnp.float32),
        mesh=mesh,
        scratch_types=[
            pltpu.VMEM((N_PAD,), jnp.float32),
            pltpu.VMEM((CHUNK,), jnp.int32),
            pltpu.VMEM((CHUNK,), jnp.int32),
            pltpu.VMEM((N_ACC,), jnp.float32),
            pltpu.VMEM_SHARED((NS, N_ACC), jnp.float32),
            pltpu.VMEM((NS * (N_ACC // NS),), jnp.float32),
            pltpu.VMEM((N_ACC // NS,), jnp.float32),
            pltpu.SemaphoreType.DMA,
            pltpu.SemaphoreType.DMA,
        ],
        compiler_params=cp,
    )(_sc_gather_segsum_body)


def _sc_gather_segsum_body(hw_hbm, src_hbm, seg_hbm, out_hbm,
                           hw_v, src_v, seg_v, acc_v, shared, tmp_v, red_v,
                           sem_hw, sem_edge):
    sid = lax.axis_index("s")
    cid = lax.axis_index("c")
    wid = sid * NC + cid
    base = wid * CHUNK
    cp_hw = pltpu.async_copy(hw_hbm, hw_v, sem_hw)
    edge_cps = []
    for j in range(NSUB):
        g = wid * NSUB + j
        row = g // (E_PER_TYPE // SUB)
        col = (g % (E_PER_TYPE // SUB)) * SUB
        edge_cps.append(pltpu.async_copy(
            src_hbm.at[row, pl.ds(col, SUB)], src_v.at[pl.ds(j * SUB, SUB)],
            sem_edge))
        edge_cps.append(pltpu.async_copy(
            seg_hbm.at[row, pl.ds(col, SUB)], seg_v.at[pl.ds(j * SUB, SUB)],
            sem_edge))

    zeros = jnp.zeros((LANES,), jnp.float32)

    @pl.loop(0, N_ACC, step=LANES)
    def _(i):
        acc_v[pl.ds(i, LANES)] = zeros

    for cp in edge_cps:
        cp.wait()
    cp_hw.wait()

    lane = lax.iota(jnp.int32, LANES)
    shift = jnp.minimum(lane + 1, LANES - 1)
    last = lane == LANES - 1
    notlast = lane < LANES - 1

    @plsc.parallel_loop(0, CHUNK, step=LANES, unroll=8)
    def _(i):
        idx = src_v[pl.ds(i, LANES)]
        segv = seg_v[pl.ds(i, LANES)]
        seg_next = plsc.load_gather(seg_v, [i + shift])
        vals = plsc.load_gather(hw_v, [idx])
        cs = plsc.cumsum(vals)
        t = (base + i) // E_PER_TYPE
        ids = segv * N_SLOT + t
        ids_next = seg_next * N_SLOT + t
        is_end = jnp.logical_or(segv != seg_next, last)
        m2 = jnp.logical_and(is_end, notlast)
        plsc.addupdate_scatter(acc_v, [ids], cs, mask=is_end)
        plsc.addupdate_scatter(acc_v, [ids_next], -cs, mask=m2)

    SLICE = N_ACC // NS
    pltpu.sync_copy(acc_v, shared.at[sid])
    plsc.subcore_barrier()
    cps = []
    for k in range(NS):
        cps.append(pltpu.async_copy(
            shared.at[k, pl.ds(sid * SLICE, SLICE)],
            tmp_v.at[pl.ds(k * SLICE, SLICE)], sem_edge))
    for cp in cps:
        cp.wait()

    @pl.loop(0, SLICE, step=LANES)
    def _(l):
        s16 = tmp_v[pl.ds(l, LANES)]
        for k in range(1, NS):
            s16 = s16 + tmp_v[pl.ds(k * SLICE + l, LANES)]
        red_v[pl.ds(l, LANES)] = s16

    pltpu.sync_copy(red_v, out_hbm.at[cid, pl.ds(sid * SLICE, SLICE)])


def _finalize_body(p_ref, m_ref, o_ref):
    s = p_ref[0] + p_ref[1]
    sm = s.reshape(BATCH, N_SLOT)[:, :N_ETYPES]
    masked = jnp.where(m_ref[...] != 0, jnp.float32(-1000000000.0), sm)
    mx = jnp.max(masked, axis=1, keepdims=True)
    e = jnp.exp(masked - mx)
    o_ref[...] = e / jnp.sum(e, axis=1, keepdims=True)


def _finalize(partials, maskf):
    return pl.pallas_call(
        _finalize_body,
        out_shape=jax.ShapeDtypeStruct((BATCH, N_ETYPES), jnp.float32),
    )(partials, maskf)


@jax.jit
def _impl(h, edge_src, seg_ids, maskf, W):
    hw = _matvec(h, W)
    src = edge_src.astype(jnp.int32)
    seg = seg_ids.astype(jnp.int32)
    partials = _get_sc_kernel()(hw, src, seg)
    return _finalize(partials, maskf)


def kernel(h, edge_src, seg_ids, mask_mat, W):
    return _impl(h, edge_src, seg_ids, mask_mat.astype(jnp.float32), W)


# speedup vs baseline: 1.0181x; 1.0086x over previous
import dataclasses
import functools

import jax
import jax.numpy as jnp
from jax import lax
from jax.experimental import pallas as pl
from jax.experimental.pallas import tpu as pltpu
from jax.experimental.pallas import tpu_sc as plsc

N_BOND = 50000
N_ETYPES = 36
E_PER_TYPE = 2048
BATCH = 64
FEAT = 768
N_EDGES = N_ETYPES * E_PER_TYPE
N_SLOT = 128
N_ACC = BATCH * N_SLOT

ROWS_BLK = 5120
N_BLKS = 10
N_PAD = ROWS_BLK * N_BLKS


def _matvec_body(h_ref, w_ref, out_ref):
    out_ref[...] = jnp.sum(h_ref[...] * w_ref[...], axis=1)


def _matvec(h, w):
    return pl.pallas_call(
        _matvec_body,
        grid=(N_BLKS,),
        in_specs=[
            pl.BlockSpec((ROWS_BLK, FEAT), lambda i: (i, 0)),
            pl.BlockSpec((1, FEAT), lambda i: (0, 0)),
        ],
        out_specs=pl.BlockSpec((ROWS_BLK,), lambda i: (i,)),
        out_shape=jax.ShapeDtypeStruct((N_PAD,), jnp.float32),
    )(h, w)


NC = 2
NS = 16
NW = NC * NS
CHUNK = N_EDGES // NW
SUB = 256
NSUB = CHUNK // SUB
LANES = 16


@functools.cache
def _get_sc_kernel():
    mesh = plsc.VectorSubcoreMesh(core_axis_name="c", subcore_axis_name="s")
    cp = pltpu.CompilerParams()
    if "needs_layout_passes" in pltpu.CompilerParams.__dataclass_fields__:
        cp = dataclasses.replace(cp, needs_layout_passes=False)
    return functools.partial(
        pl.kernel,
        out_type=jax.ShapeDtypeStruct((NW, N_ACC), jnp.float32),
        mesh=mesh,
        scratch_types=[
            pltpu.VMEM((N_PAD,), jnp.float32),
            pltpu.VMEM((CHUNK,), jnp.int32),
            pltpu.VMEM((CHUNK,), jnp.int32),
            pltpu.VMEM((N_ACC,), jnp.float32),
            pltpu.SemaphoreType.DMA,
            pltpu.SemaphoreType.DMA,
        ],
        compiler_params=cp,
    )(_sc_gather_segsum_body)


def _sc_gather_segsum_body(hw_hbm, src_hbm, seg_hbm, out_hbm,
                           hw_v, src_v, seg_v, acc_v, sem_hw, sem_edge):
    wid = lax.axis_index("s") * NC + lax.axis_index("c")
    base = wid * CHUNK
    cp_hw = pltpu.async_copy(hw_hbm, hw_v, sem_hw)
    edge_cps = []
    for j in range(NSUB):
        g = wid * NSUB + j
        row = g // (E_PER_TYPE // SUB)
        col = (g % (E_PER_TYPE // SUB)) * SUB
        edge_cps.append(pltpu.async_copy(
            src_hbm.at[row, pl.ds(col, SUB)], src_v.at[pl.ds(j * SUB, SUB)],
            sem_edge))
        edge_cps.append(pltpu.async_copy(
            seg_hbm.at[row, pl.ds(col, SUB)], seg_v.at[pl.ds(j * SUB, SUB)],
            sem_edge))

    zeros = jnp.zeros((LANES,), jnp.float32)

    @pl.loop(0, N_ACC, step=LANES)
    def _(i):
        acc_v[pl.ds(i, LANES)] = zeros

    for cp in edge_cps:
        cp.wait()
    cp_hw.wait()

    lane = lax.iota(jnp.int32, LANES)
    shift = jnp.minimum(lane + 1, LANES - 1)
    last = lane == LANES - 1
    notlast = lane < LANES - 1

    @plsc.parallel_loop(0, CHUNK, step=LANES, unroll=8)
    def _(i):
        idx = src_v[pl.ds(i, LANES)]
        segv = seg_v[pl.ds(i, LANES)]
        seg_next = plsc.load_gather(seg_v, [i + shift])
        vals = plsc.load_gather(hw_v, [idx])
        cs = plsc.cumsum(vals)
        t = (base + i) // E_PER_TYPE
        ids = segv * N_SLOT + t
        ids_next = seg_next * N_SLOT + t
        is_end = jnp.logical_or(segv != seg_next, last)
        m2 = jnp.logical_and(is_end, notlast)
        plsc.addupdate_scatter(acc_v, [ids], cs, mask=is_end)
        plsc.addupdate_scatter(acc_v, [ids_next], -cs, mask=m2)

    pltpu.sync_copy(acc_v, out_hbm.at[wid])


def _finalize_body(p_ref, m_ref, o_ref):
    s = p_ref[0]
    for k in range(1, NW):
        s = s + p_ref[k]
    sm = s.reshape(BATCH, N_SLOT)[:, :N_ETYPES]
    masked = jnp.where(m_ref[...] != 0, jnp.float32(-1000000000.0), sm)
    mx = jnp.max(masked, axis=1, keepdims=True)
    e = jnp.exp(masked - mx)
    o_ref[...] = e / jnp.sum(e, axis=1, keepdims=True)


def _finalize(partials, maskf):
    return pl.pallas_call(
        _finalize_body,
        out_shape=jax.ShapeDtypeStruct((BATCH, N_ETYPES), jnp.float32),
    )(partials, maskf)


@jax.jit
def _impl(h, edge_src, seg_ids, maskf, W):
    hw = _matvec(h, W)
    src = edge_src.astype(jnp.int32)
    seg = seg_ids.astype(jnp.int32)
    partials = _get_sc_kernel()(hw, src, seg)
    return _finalize(partials, maskf)


def kernel(h, edge_src, seg_ids, mask_mat, W):
    return _impl(h, edge_src, seg_ids, mask_mat.astype(jnp.float32), W)


# speedup vs baseline: 1.0541x; 1.0354x over previous
import dataclasses
import functools

import jax
import jax.numpy as jnp
from jax import lax
from jax.experimental import pallas as pl
from jax.experimental.pallas import tpu as pltpu
from jax.experimental.pallas import tpu_sc as plsc

N_BOND = 50000
N_ETYPES = 36
E_PER_TYPE = 2048
BATCH = 64
FEAT = 768
N_EDGES = N_ETYPES * E_PER_TYPE
N_SLOT = 128
N_ACC = BATCH * N_SLOT

ROWS_BLK = 5120
N_BLKS = 10
N_PAD = ROWS_BLK * N_BLKS


def _matvec_body(h_ref, w_ref, out_ref):
    out_ref[...] = jnp.sum(h_ref[...] * w_ref[...], axis=1)


def _matvec(h, w):
    return pl.pallas_call(
        _matvec_body,
        grid=(N_BLKS,),
        in_specs=[
            pl.BlockSpec((ROWS_BLK, FEAT), lambda i: (i, 0)),
            pl.BlockSpec((1, FEAT), lambda i: (0, 0)),
        ],
        out_specs=pl.BlockSpec((ROWS_BLK,), lambda i: (i,)),
        out_shape=jax.ShapeDtypeStruct((N_PAD,), jnp.float32),
    )(h, w)


NC = 2
NS = 16
NW = NC * NS
CHUNK = N_EDGES // NW
SUB = 256
NSUB = CHUNK // SUB
LANES = 16


@functools.cache
def _get_sc_kernel():
    mesh = plsc.VectorSubcoreMesh(core_axis_name="c", subcore_axis_name="s")
    cp = pltpu.CompilerParams()
    if "needs_layout_passes" in pltpu.CompilerParams.__dataclass_fields__:
        cp = dataclasses.replace(cp, needs_layout_passes=False)
    return functools.partial(
        pl.kernel,
        out_type=jax.ShapeDtypeStruct((NW, N_ACC), jnp.float32),
        mesh=mesh,
        scratch_types=[
            pltpu.VMEM((CHUNK,), jnp.float32),
            pltpu.VMEM((CHUNK,), jnp.int32),
            pltpu.VMEM((CHUNK,), jnp.int32),
            pltpu.VMEM((N_ACC,), jnp.float32),
            pltpu.SemaphoreType.DMA,
            pltpu.SemaphoreType.DMA,
        ],
        compiler_params=cp,
    )(_sc_gather_segsum_body)


def _sc_gather_segsum_body(hw_hbm, src_hbm, seg_hbm, out_hbm,
                           vals_v, src_v, seg_v, acc_v, sem_hw, sem_edge):
    wid = lax.axis_index("s") * NC + lax.axis_index("c")
    base = wid * CHUNK
    edge_cps = []
    for j in range(NSUB):
        g = wid * NSUB + j
        row = g // (E_PER_TYPE // SUB)
        col = (g % (E_PER_TYPE // SUB)) * SUB
        edge_cps.append(pltpu.async_copy(
            src_hbm.at[row, pl.ds(col, SUB)], src_v.at[pl.ds(j * SUB, SUB)],
            sem_edge))
        edge_cps.append(pltpu.async_copy(
            seg_hbm.at[row, pl.ds(col, SUB)], seg_v.at[pl.ds(j * SUB, SUB)],
            sem_edge))

    for cp in edge_cps:
        cp.wait()
    cp_hw = pltpu.async_copy(hw_hbm.at[src_v], vals_v, sem_hw)

    zeros = jnp.zeros((LANES,), jnp.float32)

    @pl.loop(0, N_ACC, step=LANES)
    def _(i):
        acc_v[pl.ds(i, LANES)] = zeros

    cp_hw.wait()

    lane = lax.iota(jnp.int32, LANES)
    shift = jnp.minimum(lane + 1, LANES - 1)
    last = lane == LANES - 1
    notlast = lane < LANES - 1

    @plsc.parallel_loop(0, CHUNK, step=LANES, unroll=8)
    def _(i):
        segv = seg_v[pl.ds(i, LANES)]
        seg_next = plsc.load_gather(seg_v, [i + shift])
        vals = vals_v[pl.ds(i, LANES)]
        cs = plsc.cumsum(vals)
        t = (base + i) // E_PER_TYPE
        ids = segv * N_SLOT + t
        ids_next = seg_next * N_SLOT + t
        is_end = jnp.logical_or(segv != seg_next, last)
        m2 = jnp.logical_and(is_end, notlast)
        plsc.addupdate_scatter(acc_v, [ids], cs, mask=is_end)
        plsc.addupdate_scatter(acc_v, [ids_next], -cs, mask=m2)

    pltpu.sync_copy(acc_v, out_hbm.at[wid])


def _finalize_body(p_ref, m_ref, o_ref):
    s = p_ref[0]
    for k in range(1, NW):
        s = s + p_ref[k]
    sm = s.reshape(BATCH, N_SLOT)[:, :N_ETYPES]
    masked = jnp.where(m_ref[...] != 0, jnp.float32(-1000000000.0), sm)
    mx = jnp.max(masked, axis=1, keepdims=True)
    e = jnp.exp(masked - mx)
    o_ref[...] = e / jnp.sum(e, axis=1, keepdims=True)


def _finalize(partials, maskf):
    return pl.pallas_call(
        _finalize_body,
        out_shape=jax.ShapeDtypeStruct((BATCH, N_ETYPES), jnp.float32),
    )(partials, maskf)


@jax.jit
def _impl(h, edge_src, seg_ids, maskf, W):
    hw = _matvec(h, W)
    src = edge_src.astype(jnp.int32)
    seg = seg_ids.astype(jnp.int32)
    partials = _get_sc_kernel()(hw, src, seg)
    return _finalize(partials, maskf)


def kernel(h, edge_src, seg_ids, mask_mat, W):
    return _impl(h, edge_src, seg_ids, mask_mat.astype(jnp.float32), W)
